# baseline (speedup 1.0000x reference)
import functools

import jax
import jax.numpy as jnp
from jax import lax
from jax.experimental import pallas as pl
from jax.experimental.pallas import tpu as pltpu
from jax.experimental.pallas import tpu_sc as plsc


def _tail_tc(table):
    V, D = table.shape
    R = 10000

    def body(i_ref, o_ref):
        o_ref[...] = i_ref[...]

    return pl.pallas_call(
        body,
        grid=(V // R,),
        in_specs=[pl.BlockSpec((R, 128), lambda i: (i, 7))],
        out_specs=pl.BlockSpec((R, 128), lambda i: (i, 0)),
        out_shape=jax.ShapeDtypeStruct((V, 128), jnp.float32),
    )(table)


def _gather_sc(table, tail, idx):
    B = idx.shape[0]
    V, D = table.shape
    NB = D // 128
    Dp = (NB + 1) * 128
    info = plsc.get_sparse_core_info()
    NC, NS = info.num_cores, info.num_subcores
    NW = NC * NS
    b_per_w = B // NW
    C = 32
    n_chunks = b_per_w // C

    mesh = plsc.VectorSubcoreMesh(core_axis_name="c", subcore_axis_name="s")

    @functools.partial(
        pl.kernel,
        mesh=mesh,
        out_type=jax.ShapeDtypeStruct((B, Dp), jnp.float32),
        scratch_types=[
            pltpu.VMEM((b_per_w,), jnp.int32),
            pltpu.VMEM((C, Dp), jnp.float32),
            pltpu.VMEM((C, Dp), jnp.float32),
            pltpu.SemaphoreType.DMA,
            pltpu.SemaphoreType.DMA,
            pltpu.SemaphoreType.DMA,
            pltpu.SemaphoreType.DMA,
        ],
        compiler_params=pltpu.CompilerParams(use_tc_tiling_on_sc=True),
    )
    def k(table_hbm, tail_hbm, idx_hbm, out_hbm, idx_v, rows0, rows1, g0, g1, w0, w1):
        wid = lax.axis_index("s") * NC + lax.axis_index("c")
        base = wid * b_per_w
        pltpu.sync_copy(idx_hbm.at[pl.ds(base, b_per_w)], idx_v)

        bufs = (rows0, rows1)
        gsem = (g0, g1)
        wsem = (w0, w1)

        def gather(g):
            b = g & 1
            ids = idx_v.at[pl.ds(g * C, C)]
            cps = []
            for j in range(NB):
                cps.append(
                    pltpu.async_copy(
                        table_hbm.at[ids, pl.ds(j * 128, 128)],
                        bufs[b].at[:, pl.ds(j * 128, 128)],
                        gsem[b],
                    )
                )
            cps.append(
                pltpu.async_copy(
                    tail_hbm.at[ids],
                    bufs[b].at[:, pl.ds(NB * 128, 128)],
                    gsem[b],
                )
            )
            return cps

        gathers = [gather(0), gather(1)]
        writes = [None] * n_chunks
        for g in range(n_chunks):
            b = g & 1
            for cp in gathers[g]:
                cp.wait()
            writes[g] = pltpu.async_copy(
                bufs[b], out_hbm.at[pl.ds(base + g * C, C)], wsem[b]
            )
            if g + 2 < n_chunks:
                writes[g].wait()
                gathers.append(gather(g + 2))
        writes[n_chunks - 2].wait()
        writes[n_chunks - 1].wait()

    return k(table, tail, idx)


def kernel(x, index, logits_table):
    del x
    D = logits_table.shape[1]
    tail = _tail_tc(logits_table)
    out_p = _gather_sc(logits_table, tail, index.astype(jnp.int32))
    return out_p[:, :D]


# speedup vs baseline: 1.3221x; 1.3221x over previous
import functools

import jax
import jax.numpy as jnp
from jax import lax
from jax.experimental import pallas as pl
from jax.experimental.pallas import tpu as pltpu
from jax.experimental.pallas import tpu_sc as plsc

_BT = 1024


def _transpose_pad_tc(tableT, Vp):
    D, V = tableT.shape

    def body(i_ref, o_ref):
        o_ref[...] = i_ref[...].T

    return pl.pallas_call(
        body,
        grid=(Vp // _BT,),
        in_specs=[pl.BlockSpec((_BT, _BT), lambda j: (0, j))],
        out_specs=pl.BlockSpec((_BT, _BT), lambda j: (j, 0)),
        out_shape=jax.ShapeDtypeStruct((Vp, _BT), jnp.float32),
    )(tableT)


def _gather_sc(table_p, idx):
    B = idx.shape[0]
    Vp, Dp = table_p.shape
    NB = Dp // 128
    info = plsc.get_sparse_core_info()
    NC, NS = info.num_cores, info.num_subcores
    NW = NC * NS
    b_per_w = B // NW
    C = 32
    n_chunks = b_per_w // C

    mesh = plsc.VectorSubcoreMesh(core_axis_name="c", subcore_axis_name="s")

    @functools.partial(
        pl.kernel,
        mesh=mesh,
        out_type=jax.ShapeDtypeStruct((B, Dp), jnp.float32),
        scratch_types=[
            pltpu.VMEM((b_per_w,), jnp.int32),
            pltpu.VMEM((C, Dp), jnp.float32),
            pltpu.VMEM((C, Dp), jnp.float32),
            pltpu.SemaphoreType.DMA,
            pltpu.SemaphoreType.DMA,
            pltpu.SemaphoreType.DMA,
            pltpu.SemaphoreType.DMA,
        ],
        compiler_params=pltpu.CompilerParams(use_tc_tiling_on_sc=True),
    )
    def k(table_hbm, idx_hbm, out_hbm, idx_v, rows0, rows1, g0, g1, w0, w1):
        wid = lax.axis_index("s") * NC + lax.axis_index("c")
        base = wid * b_per_w
        pltpu.sync_copy(idx_hbm.at[pl.ds(base, b_per_w)], idx_v)

        bufs = (rows0, rows1)
        gsem = (g0, g1)
        wsem = (w0, w1)

        def gather(g):
            b = g & 1
            ids = idx_v.at[pl.ds(g * C, C)]
            return [
                pltpu.async_copy(
                    table_hbm.at[ids, pl.ds(j * 128, 128)],
                    bufs[b].at[:, pl.ds(j * 128, 128)],
                    gsem[b],
                )
                for j in range(NB)
            ]

        gathers = [gather(0), gather(1)]
        writes = [None] * n_chunks
        for g in range(n_chunks):
            b = g & 1
            for cp in gathers[g]:
                cp.wait()
            writes[g] = pltpu.async_copy(
                bufs[b], out_hbm.at[pl.ds(base + g * C, C)], wsem[b]
            )
            if g + 2 < n_chunks:
                writes[g].wait()
                gathers.append(gather(g + 2))
        writes[n_chunks - 2].wait()
        writes[n_chunks - 1].wait()

    return k(table_p, idx)


def kernel(x, index, logits_table):
    del x
    V, D = logits_table.shape
    Vp = pl.cdiv(V, _BT) * _BT
    table_p = _transpose_pad_tc(logits_table.T, Vp)
    out_p = _gather_sc(table_p, index.astype(jnp.int32))
    return out_p[:, :D]
